# baseline (device time: 15417 ns/iter reference)
import jax
import jax.numpy as jnp
from jax import lax
from jax.experimental import pallas as pl
from jax.experimental.pallas import tpu as pltpu

T = 256
V_SHARD = 4096
G = 8
VC = V_SHARD // G


def kernel(x, W, labels):
    def body(x_ref, w_ref, labels_ref, out_ref,
             x_bf, send_buf, recv_buf, send_sem, recv_sem):
        i = pl.program_id(0)
        my_x = lax.axis_index("x")
        my_y = lax.axis_index("y")
        my_z = lax.axis_index("z")
        partner = (1 - my_x, my_y, my_z)
        barrier = pltpu.get_barrier_semaphore()

        @pl.when(i == 0)
        def _():
            pl.semaphore_signal(barrier, inc=1, device_id=partner,
                                device_id_type=pl.DeviceIdType.MESH)
            x_bf[...] = x_ref[...].astype(jnp.bfloat16)
            send_buf[...] = jnp.zeros((2, T), jnp.float32)

        lt = lax.dot_general(
            w_ref[...].astype(jnp.bfloat16),
            x_bf[...],
            dimension_numbers=(((0,), (1,)), ((), ())),
            preferred_element_type=jnp.float32,
        )
        rows = (lax.broadcasted_iota(jnp.int32, (VC, T), 0)
                + my_x * V_SHARD + i * VC)
        masked = jnp.where(rows == labels_ref[...], lt, 0.0)
        ones = jnp.ones((1, VC), jnp.float32)
        s = lax.dot_general(ones, jnp.exp(lt),
                            dimension_numbers=(((1,), (0,)), ((), ())),
                            preferred_element_type=jnp.float32)
        tgt = lax.dot_general(ones, masked,
                              dimension_numbers=(((1,), (0,)), ((), ())),
                              preferred_element_type=jnp.float32)
        send_buf[0:1, :] += s
        send_buf[1:2, :] += tgt

        @pl.when(i == G - 1)
        def _():
            pl.semaphore_wait(barrier, 1)
            rdma = pltpu.make_async_remote_copy(
                src_ref=send_buf,
                dst_ref=recv_buf,
                send_sem=send_sem,
                recv_sem=recv_sem,
                device_id=partner,
                device_id_type=pl.DeviceIdType.MESH,
            )
            rdma.start()
            rdma.wait()
            out_ref[...] = (jnp.log(send_buf[0:1, :] + recv_buf[0:1, :])
                            - (send_buf[1:2, :] + recv_buf[1:2, :]))

    out = pl.pallas_call(
        body,
        grid=(G,),
        out_shape=jax.ShapeDtypeStruct((1, T), jnp.float32),
        in_specs=[
            pl.BlockSpec((T, 512), lambda i: (0, 0),
                         memory_space=pltpu.VMEM),
            pl.BlockSpec((512, VC), lambda i: (0, i),
                         memory_space=pltpu.VMEM),
            pl.BlockSpec((1, T), lambda i: (0, 0),
                         memory_space=pltpu.VMEM),
        ],
        out_specs=pl.BlockSpec((1, T), lambda i: (0, 0),
                               memory_space=pltpu.VMEM),
        scratch_shapes=[
            pltpu.VMEM((T, 512), jnp.bfloat16),
            pltpu.VMEM((2, T), jnp.float32),
            pltpu.VMEM((2, T), jnp.float32),
            pltpu.SemaphoreType.DMA,
            pltpu.SemaphoreType.DMA,
        ],
        compiler_params=pltpu.CompilerParams(
            collective_id=0,
            dimension_semantics=("arbitrary",),
        ),
    )(x, W, labels.reshape(1, T))
    return out.reshape(T)


# device time: 15166 ns/iter; 1.0166x vs baseline; 1.0166x over previous
import jax
import jax.numpy as jnp
from jax import lax
from jax.experimental import pallas as pl
from jax.experimental.pallas import tpu as pltpu

T = 256
D = 512
V_SHARD = 4096
G = 8
VC = V_SHARD // G


def kernel(x, W, labels):
    def body(x_ref, w_hbm, labels_ref, out_ref,
             w_buf, send_buf, recv_buf, copy_sems, send_sem, recv_sem):
        my_x = lax.axis_index("x")
        my_y = lax.axis_index("y")
        my_z = lax.axis_index("z")
        partner = (1 - my_x, my_y, my_z)

        barrier = pltpu.get_barrier_semaphore()
        pl.semaphore_signal(barrier, inc=1, device_id=partner,
                            device_id_type=pl.DeviceIdType.MESH)

        def chunk_copy(c, slot):
            return pltpu.make_async_copy(
                w_hbm.at[:, pl.ds(c * VC, VC)],
                w_buf.at[slot],
                copy_sems.at[slot],
            )

        chunk_copy(0, 0).start()
        x_bf = x_ref[...].astype(jnp.bfloat16)

        s_acc = jnp.zeros((1, T), jnp.float32)
        t_acc = jnp.zeros((1, T), jnp.float32)
        ones = jnp.ones((1, VC), jnp.float32)
        for c in range(G):
            slot = c % 2
            chunk_copy(c, slot).wait()
            if c + 1 < G:
                chunk_copy(c + 1, 1 - slot).start()
            lt = lax.dot_general(
                w_buf[slot].astype(jnp.bfloat16),
                x_bf,
                dimension_numbers=(((0,), (1,)), ((), ())),
                preferred_element_type=jnp.float32,
            )
            rows = (lax.broadcasted_iota(jnp.int32, (VC, T), 0)
                    + my_x * V_SHARD + c * VC)
            masked = jnp.where(rows == labels_ref[...], lt, 0.0)
            s_acc += lax.dot_general(
                ones, jnp.exp(lt),
                dimension_numbers=(((1,), (0,)), ((), ())),
                preferred_element_type=jnp.float32)
            t_acc += lax.dot_general(
                ones, masked,
                dimension_numbers=(((1,), (0,)), ((), ())),
                preferred_element_type=jnp.float32)

        send_buf[0:1, :] = s_acc
        send_buf[1:2, :] = t_acc
        pl.semaphore_wait(barrier, 1)
        rdma = pltpu.make_async_remote_copy(
            src_ref=send_buf,
            dst_ref=recv_buf,
            send_sem=send_sem,
            recv_sem=recv_sem,
            device_id=partner,
            device_id_type=pl.DeviceIdType.MESH,
        )
        rdma.start()
        rdma.wait()
        out_ref[...] = (jnp.log(s_acc[0:1, :] + recv_buf[0:1, :])
                        - (t_acc[0:1, :] + recv_buf[1:2, :]))

    out = pl.pallas_call(
        body,
        out_shape=jax.ShapeDtypeStruct((1, T), jnp.float32),
        in_specs=[
            pl.BlockSpec(memory_space=pltpu.VMEM),
            pl.BlockSpec(memory_space=pltpu.MemorySpace.HBM),
            pl.BlockSpec(memory_space=pltpu.VMEM),
        ],
        out_specs=pl.BlockSpec(memory_space=pltpu.VMEM),
        scratch_shapes=[
            pltpu.VMEM((2, D, VC), jnp.float32),
            pltpu.VMEM((2, T), jnp.float32),
            pltpu.VMEM((2, T), jnp.float32),
            pltpu.SemaphoreType.DMA((2,)),
            pltpu.SemaphoreType.DMA,
            pltpu.SemaphoreType.DMA,
        ],
        compiler_params=pltpu.CompilerParams(collective_id=0),
    )(x, W, labels.reshape(1, T))
    return out.reshape(T)


# device time: 12965 ns/iter; 1.1891x vs baseline; 1.1698x over previous
import jax
import jax.numpy as jnp
from jax import lax
from jax.experimental import pallas as pl
from jax.experimental.pallas import tpu as pltpu

T = 256
D = 512
V_SHARD = 4096
KC = D // 2


def kernel(x, W, labels):
    def body(x_ref, w_hbm, labels_ref, out_ref,
             w_buf, send_buf, recv_buf, copy_sems, send_sem, recv_sem):
        my_x = lax.axis_index("x")
        my_y = lax.axis_index("y")
        my_z = lax.axis_index("z")
        partner = (1 - my_x, my_y, my_z)

        barrier = pltpu.get_barrier_semaphore()
        pl.semaphore_signal(barrier, inc=1, device_id=partner,
                            device_id_type=pl.DeviceIdType.MESH)

        cps = [
            pltpu.make_async_copy(
                w_hbm.at[pl.ds(c * KC, KC), :], w_buf.at[c], copy_sems.at[c])
            for c in range(2)
        ]
        cps[0].start()
        cps[1].start()
        x_bf = x_ref[...].astype(jnp.bfloat16)

        cps[0].wait()
        lt0 = lax.dot_general(
            w_buf[0].astype(jnp.bfloat16), x_bf[:, 0:KC],
            dimension_numbers=(((0,), (1,)), ((), ())),
            preferred_element_type=jnp.float32,
        )
        cps[1].wait()
        lt = lt0 + lax.dot_general(
            w_buf[1].astype(jnp.bfloat16), x_bf[:, KC:D],
            dimension_numbers=(((0,), (1,)), ((), ())),
            preferred_element_type=jnp.float32,
        )

        rows = lax.broadcasted_iota(jnp.int32, (V_SHARD, T), 0) + my_x * V_SHARD
        masked = jnp.where(rows == labels_ref[...], lt, 0.0)
        ones = jnp.ones((1, V_SHARD), jnp.float32)
        s = lax.dot_general(ones, jnp.exp(lt),
                            dimension_numbers=(((1,), (0,)), ((), ())),
                            preferred_element_type=jnp.float32)
        tgt = lax.dot_general(ones, masked,
                              dimension_numbers=(((1,), (0,)), ((), ())),
                              preferred_element_type=jnp.float32)
        send_buf[0:1, :] = s
        send_buf[1:2, :] = tgt

        pl.semaphore_wait(barrier, 1)
        rdma = pltpu.make_async_remote_copy(
            src_ref=send_buf,
            dst_ref=recv_buf,
            send_sem=send_sem,
            recv_sem=recv_sem,
            device_id=partner,
            device_id_type=pl.DeviceIdType.MESH,
        )
        rdma.start()
        rdma.wait()
        out_ref[...] = (jnp.log(s + recv_buf[0:1, :])
                        - (tgt + recv_buf[1:2, :]))

    out = pl.pallas_call(
        body,
        out_shape=jax.ShapeDtypeStruct((1, T), jnp.float32),
        in_specs=[
            pl.BlockSpec(memory_space=pltpu.VMEM),
            pl.BlockSpec(memory_space=pltpu.MemorySpace.HBM),
            pl.BlockSpec(memory_space=pltpu.VMEM),
        ],
        out_specs=pl.BlockSpec(memory_space=pltpu.VMEM),
        scratch_shapes=[
            pltpu.VMEM((2, KC, V_SHARD), jnp.float32),
            pltpu.VMEM((2, T), jnp.float32),
            pltpu.VMEM((2, T), jnp.float32),
            pltpu.SemaphoreType.DMA((2,)),
            pltpu.SemaphoreType.DMA,
            pltpu.SemaphoreType.DMA,
        ],
        compiler_params=pltpu.CompilerParams(collective_id=0),
    )(x, W, labels.reshape(1, T))
    return out.reshape(T)


# device time: 12028 ns/iter; 1.2818x vs baseline; 1.0779x over previous
import jax
import jax.numpy as jnp
from jax import lax
from jax.experimental import pallas as pl
from jax.experimental.pallas import tpu as pltpu

T = 256
V_SHARD = 4096


def kernel(x, W, labels):
    def body(x_ref, w_ref, labels_ref, out_ref,
             send_buf, recv_buf, send_sem, recv_sem):
        my_x = lax.axis_index("x")
        my_y = lax.axis_index("y")
        my_z = lax.axis_index("z")
        partner = (1 - my_x, my_y, my_z)

        barrier = pltpu.get_barrier_semaphore()
        pl.semaphore_signal(barrier, inc=1, device_id=partner,
                            device_id_type=pl.DeviceIdType.MESH)

        lt = lax.dot_general(
            w_ref[...], x_ref[...],
            dimension_numbers=(((0,), (1,)), ((), ())),
            preferred_element_type=jnp.float32,
        )
        rows = lax.broadcasted_iota(jnp.int32, (V_SHARD, T), 0) + my_x * V_SHARD
        masked = jnp.where(rows == labels_ref[...], lt, 0.0)
        ones = jnp.ones((1, V_SHARD), jnp.float32)
        s = lax.dot_general(ones, jnp.exp(lt),
                            dimension_numbers=(((1,), (0,)), ((), ())),
                            preferred_element_type=jnp.float32)
        tgt = lax.dot_general(ones, masked,
                              dimension_numbers=(((1,), (0,)), ((), ())),
                              preferred_element_type=jnp.float32)
        send_buf[0:1, :] = s
        send_buf[1:2, :] = tgt

        pl.semaphore_wait(barrier, 1)
        rdma = pltpu.make_async_remote_copy(
            src_ref=send_buf,
            dst_ref=recv_buf,
            send_sem=send_sem,
            recv_sem=recv_sem,
            device_id=partner,
            device_id_type=pl.DeviceIdType.MESH,
        )
        rdma.start()
        rdma.wait()

        out_ref[...] = (jnp.log(s + recv_buf[0:1, :])
                        - (tgt + recv_buf[1:2, :]))

    out = pl.pallas_call(
        body,
        out_shape=jax.ShapeDtypeStruct((1, T), jnp.float32),
        in_specs=[
            pl.BlockSpec(memory_space=pltpu.VMEM),
            pl.BlockSpec(memory_space=pltpu.VMEM),
            pl.BlockSpec(memory_space=pltpu.VMEM),
        ],
        out_specs=pl.BlockSpec(memory_space=pltpu.VMEM),
        scratch_shapes=[
            pltpu.VMEM((2, T), jnp.float32),
            pltpu.VMEM((2, T), jnp.float32),
            pltpu.SemaphoreType.DMA,
            pltpu.SemaphoreType.DMA,
        ],
        compiler_params=pltpu.CompilerParams(collective_id=0),
    )(x, W, labels.reshape(1, T))
    return out.reshape(T)
